# Initial kernel scaffold; baseline (speedup 1.0000x reference)
import jax
import jax.numpy as jnp
from jax.experimental import pallas as pl


def kernel(x_batch, y_batch, thetas, eps, n_projections):
    raise NotImplementedError("write your pallas kernel here")


# speedup vs baseline: 8.3185x; 8.3185x over previous
import functools

import jax
import jax.numpy as jnp
import numpy as np
from jax import lax
from jax.experimental import pallas as pl
from jax.experimental.pallas import tpu as pltpu
from jax.experimental.pallas import tpu_sc as plsc

L = 16
_MININT = np.int32(-2147483648)


def _normalize(th):
    n2 = jnp.sum(th * th, axis=1, keepdims=True)
    return th / jnp.maximum(jnp.sqrt(n2), 1e-12)


def _proj_body(x_ref, y_ref, th_ref, xp_ref, yp_ref):
    th = _normalize(th_ref[...])
    dn = (((1,), (1,)), ((), ()))
    xp_ref[0] = lax.dot_general(th, x_ref[0], dn, preferred_element_type=jnp.float32)
    yp_ref[0] = lax.dot_general(th, y_ref[0], dn, preferred_element_type=jnp.float32)


def _project(x, y, thetas, bn):
    B, N, D = x.shape
    P = thetas.shape[0]
    grid = (B, N // bn)
    xy_spec = pl.BlockSpec((1, bn, D), lambda b, n: (b, n, 0))
    th_spec = pl.BlockSpec((P, D), lambda b, n: (0, 0))
    out_spec = pl.BlockSpec((1, P, bn), lambda b, n: (b, 0, n))
    shape = jax.ShapeDtypeStruct((B, P, N), jnp.float32)
    return pl.pallas_call(
        _proj_body,
        grid=grid,
        in_specs=[xy_spec, xy_spec, th_spec],
        out_specs=[out_spec, out_spec],
        out_shape=[shape, shape],
    )(x, y, thetas)


def _monotone(v):
    return jnp.where(v < 0, ~v, v ^ _MININT)


def _radix_pass(src_k, dst_k, src_v, dst_v, hist, shift, chunk, lane):
    nvec = hist.shape[0] // L

    def zero_body(i, c):
        hist[pl.ds(i * L, L)] = jnp.zeros((L,), jnp.int32)
        return c

    lax.fori_loop(0, nvec, zero_body, 0)

    ones = jnp.ones((L,), jnp.int32)

    def hist_body(i, c):
        k = plsc.load_gather(src_k, [lane * chunk + i])
        d = lax.shift_right_logical(k, shift) & 255
        plsc.addupdate_scatter(hist, [d * L + lane], ones)
        return c

    lax.fori_loop(0, chunk, hist_body, 0)

    def scan_body(i, carry):
        v = hist[pl.ds(i * L, L)]
        s = jnp.cumsum(v)
        hist[pl.ds(i * L, L)] = s - v + carry
        return carry + jnp.sum(v)

    lax.fori_loop(0, nvec, scan_body, jnp.int32(0))

    def perm_body(i, c):
        gidx = lane * chunk + i
        k = plsc.load_gather(src_k, [gidx])
        d = lax.shift_right_logical(k, shift) & 255
        hidx = d * L + lane
        o = plsc.load_gather(hist, [hidx])
        plsc.store_scatter(hist, [hidx], o + 1)
        plsc.store_scatter(dst_k, [o], k)
        if src_v is not None:
            plsc.store_scatter(dst_v, [o], plsc.load_gather(src_v, [gidx]))
        return c

    lax.fori_loop(0, chunk, perm_body, 0)


def _sc_transport_body(xp_hbm, yp_hbm, out_hbm, xb, yb, ka, kb, va, vb, hist):
    nc = 2
    wid = lax.axis_index("s") * nc + lax.axis_index("c")
    rows = xp_hbm.shape[0]
    n = xp_hbm.shape[1]
    chunk = n // L
    nvec = n // L
    lane = lax.iota(jnp.int32, L)
    rows_per_w = rows // 32

    def row_body(t, c):
        r = wid * rows_per_w + t
        pltpu.sync_copy(xp_hbm.at[r], xb)
        pltpu.sync_copy(yp_hbm.at[r], yb)

        def ymono(i, c2):
            ka[pl.ds(i * L, L)] = _monotone(lax.bitcast_convert_type(yb[pl.ds(i * L, L)], jnp.int32))
            return c2

        lax.fori_loop(0, nvec, ymono, 0)
        for p in range(4):
            s, d = (ka, kb) if p % 2 == 0 else (kb, ka)
            _radix_pass(s, d, None, None, hist, 8 * p, chunk, lane)

        def yun(i, c2):
            m = ka[pl.ds(i * L, L)]
            yb[pl.ds(i * L, L)] = lax.bitcast_convert_type(jnp.where(m < 0, m ^ _MININT, ~m), jnp.float32)
            return c2

        lax.fori_loop(0, nvec, yun, 0)

        def xmono(i, c2):
            ka[pl.ds(i * L, L)] = _monotone(lax.bitcast_convert_type(xb[pl.ds(i * L, L)], jnp.int32))
            va[pl.ds(i * L, L)] = lane + i * L
            return c2

        lax.fori_loop(0, nvec, xmono, 0)
        for p in range(4):
            s, d = (ka, kb) if p % 2 == 0 else (kb, ka)
            sv, dv = (va, vb) if p % 2 == 0 else (vb, va)
            _radix_pass(s, d, sv, dv, hist, 8 * p, chunk, lane)

        def scat(i, c2):
            idx = va[pl.ds(i * L, L)]
            val = lax.bitcast_convert_type(yb[pl.ds(i * L, L)], jnp.int32)
            plsc.store_scatter(kb, [idx], val)
            return c2

        lax.fori_loop(0, nvec, scat, 0)

        def dif(i, c2):
            sl = pl.ds(i * L, L)
            xb[sl] = lax.bitcast_convert_type(kb[sl], jnp.float32) - xb[sl]
            return c2

        lax.fori_loop(0, nvec, dif, 0)
        pltpu.sync_copy(xb, out_hbm.at[r])
        return c

    lax.fori_loop(0, rows_per_w, row_body, 0)


def _sc_transport(xp, yp):
    R, N = xp.shape
    mesh = plsc.VectorSubcoreMesh(
        core_axis_name="c", subcore_axis_name="s", num_cores=2, num_subcores=16
    )
    f = pl.kernel(
        _sc_transport_body,
        out_type=jax.ShapeDtypeStruct((R, N), jnp.float32),
        mesh=mesh,
        compiler_params=pltpu.CompilerParams(needs_layout_passes=False),
        scratch_types=[
            pltpu.VMEM((N,), jnp.float32),
            pltpu.VMEM((N,), jnp.float32),
            pltpu.VMEM((N,), jnp.int32),
            pltpu.VMEM((N,), jnp.int32),
            pltpu.VMEM((N,), jnp.int32),
            pltpu.VMEM((N,), jnp.int32),
            pltpu.VMEM((256 * L,), jnp.int32),
        ],
    )
    return f(xp, yp)


def _recomb_body(x_ref, diff_ref, th_ref, o_ref, *, inv_p):
    th = _normalize(th_ref[...])
    dn = (((0,), (0,)), ((), ()))
    contrib = lax.dot_general(diff_ref[0], th, dn, preferred_element_type=jnp.float32)
    o_ref[0] = x_ref[0] + contrib * inv_p


def _recombine(x, diff, thetas, bn):
    B, N, D = x.shape
    P = thetas.shape[0]
    grid = (B, N // bn)
    return pl.pallas_call(
        functools.partial(_recomb_body, inv_p=1.0 / P),
        grid=grid,
        in_specs=[
            pl.BlockSpec((1, bn, D), lambda b, n: (b, n, 0)),
            pl.BlockSpec((1, P, bn), lambda b, n: (b, 0, n)),
            pl.BlockSpec((P, D), lambda b, n: (0, 0)),
        ],
        out_specs=pl.BlockSpec((1, bn, D), lambda b, n: (b, n, 0)),
        out_shape=jax.ShapeDtypeStruct((B, N, D), jnp.float32),
    )(x, diff, thetas)


def kernel(x_batch, y_batch, thetas, eps, n_projections):
    B, N, D = x_batch.shape
    P = thetas.shape[0]
    bn = 2048
    xp, yp = _project(x_batch, y_batch, thetas, bn)
    diff = _sc_transport(xp.reshape(B * P, N), yp.reshape(B * P, N))
    return _recombine(x_batch, diff.reshape(B, P, N), thetas, bn)


# speedup vs baseline: 10.6547x; 1.2808x over previous
import functools

import jax
import jax.numpy as jnp
import numpy as np
from jax import lax
from jax.experimental import pallas as pl
from jax.experimental.pallas import tpu as pltpu
from jax.experimental.pallas import tpu_sc as plsc

L = 16
_MININT = np.int32(-2147483648)


def _normalize(th):
    n2 = jnp.sum(th * th, axis=1, keepdims=True)
    return th / jnp.maximum(jnp.sqrt(n2), 1e-12)


def _monotone(v):
    return jnp.where(v < 0, ~v, v ^ _MININT)


def _unmonotone_bits(m):
    return jnp.where(m < 0, m ^ _MININT, ~m)


def _proj_body(x_ref, y_ref, th_ref, xp_ref, xk_ref, yk_ref):
    th = _normalize(th_ref[...])
    dn = (((1,), (1,)), ((), ()))
    xp = lax.dot_general(th, x_ref[0], dn, preferred_element_type=jnp.float32)
    yp = lax.dot_general(th, y_ref[0], dn, preferred_element_type=jnp.float32)
    xp_ref[0] = xp
    xk_ref[0] = _monotone(lax.bitcast_convert_type(xp, jnp.int32))
    yk_ref[0] = _monotone(lax.bitcast_convert_type(yp, jnp.int32))


def _project(x, y, thetas, bn):
    B, N, D = x.shape
    P = thetas.shape[0]
    grid = (B, N // bn)
    xy_spec = pl.BlockSpec((1, bn, D), lambda b, n: (b, n, 0))
    th_spec = pl.BlockSpec((P, D), lambda b, n: (0, 0))
    out_spec = pl.BlockSpec((1, P, bn), lambda b, n: (b, 0, n))
    return pl.pallas_call(
        _proj_body,
        grid=grid,
        in_specs=[xy_spec, xy_spec, th_spec],
        out_specs=[out_spec, out_spec, out_spec],
        out_shape=[
            jax.ShapeDtypeStruct((B, P, N), jnp.float32),
            jax.ShapeDtypeStruct((B, P, N), jnp.int32),
            jax.ShapeDtypeStruct((B, P, N), jnp.int32),
        ],
    )(x, y, thetas)


def _radix_pass(src_k, dst_k, src_v, dst_v, hist, shift, chunk, lane):
    nhv = hist.shape[0] // L

    @plsc.parallel_loop(0, nhv, unroll=8)
    def _(i):
        hist[pl.ds(i * L, L)] = jnp.zeros((L,), jnp.int32)

    ones = jnp.ones((L,), jnp.int32)

    @plsc.parallel_loop(0, chunk, unroll=8)
    def _(i):
        k = plsc.load_gather(src_k, [lane * chunk + i])
        d = lax.shift_right_logical(k, shift) & 255
        plsc.addupdate_scatter(hist, [d * L + lane], ones)

    def scan_body(i, carry):
        v = hist[pl.ds(i * L, L)]
        s = jnp.cumsum(v)
        hist[pl.ds(i * L, L)] = s - v + carry
        return carry + jnp.sum(v)

    lax.fori_loop(0, nhv, scan_body, jnp.int32(0))

    def perm_one(i):
        gidx = lane * chunk + i
        k = plsc.load_gather(src_k, [gidx])
        d = lax.shift_right_logical(k, shift) & 255
        hidx = d * L + lane
        o = plsc.load_gather(hist, [hidx])
        plsc.store_scatter(hist, [hidx], o + 1)
        plsc.store_scatter(dst_k, [o], k)
        if src_v is not None:
            plsc.store_scatter(dst_v, [o], plsc.load_gather(src_v, [gidx]))

    def perm_body(i2, c):
        perm_one(i2 * 2)
        perm_one(i2 * 2 + 1)
        return c

    lax.fori_loop(0, chunk // 2, perm_body, 0)


def _sc_transport_body(xk_hbm, yk_hbm, xp_hbm, out_hbm, xb, k0, k1, v0, v1, hist):
    nc = 2
    wid = lax.axis_index("s") * nc + lax.axis_index("c")
    rows = xk_hbm.shape[0]
    n = xk_hbm.shape[1]
    chunk = n // L
    nvec = n // L
    lane = lax.iota(jnp.int32, L)
    rows_per_w = rows // 32

    def row_body(t, c):
        r = wid * rows_per_w + t
        pltpu.sync_copy(xk_hbm.at[r], k0)
        pltpu.sync_copy(xp_hbm.at[r], xb)

        @plsc.parallel_loop(0, nvec, unroll=8)
        def _(i):
            v0[pl.ds(i * L, L)] = lane + i * L

        for p in range(4):
            s, d = (k0, k1) if p % 2 == 0 else (k1, k0)
            sv, dv = (v0, v1) if p % 2 == 0 else (v1, v0)
            _radix_pass(s, d, sv, dv, hist, 8 * p, chunk, lane)

        pltpu.sync_copy(yk_hbm.at[r], k1)
        for p in range(4):
            s, d = (k1, v1) if p % 2 == 0 else (v1, k1)
            _radix_pass(s, d, None, None, hist, 8 * p, chunk, lane)

        @plsc.parallel_loop(0, nvec, unroll=4)
        def _(i):
            sl = pl.ds(i * L, L)
            plsc.store_scatter(v1, [v0[sl]], _unmonotone_bits(k1[sl]))

        @plsc.parallel_loop(0, nvec, unroll=4)
        def _(i):
            sl = pl.ds(i * L, L)
            xb[sl] = lax.bitcast_convert_type(v1[sl], jnp.float32) - xb[sl]

        pltpu.sync_copy(xb, out_hbm.at[r])
        return c

    lax.fori_loop(0, rows_per_w, row_body, 0)


def _sc_transport(xk, yk, xp):
    R, N = xk.shape
    mesh = plsc.VectorSubcoreMesh(
        core_axis_name="c", subcore_axis_name="s", num_cores=2, num_subcores=16
    )
    f = pl.kernel(
        _sc_transport_body,
        out_type=jax.ShapeDtypeStruct((R, N), jnp.float32),
        mesh=mesh,
        compiler_params=pltpu.CompilerParams(needs_layout_passes=False),
        scratch_types=[
            pltpu.VMEM((N,), jnp.float32),
            pltpu.VMEM((N,), jnp.int32),
            pltpu.VMEM((N,), jnp.int32),
            pltpu.VMEM((N,), jnp.int32),
            pltpu.VMEM((N,), jnp.int32),
            pltpu.VMEM((256 * L,), jnp.int32),
        ],
    )
    return f(xk, yk, xp)


def _recomb_body(x_ref, diff_ref, th_ref, o_ref, *, inv_p):
    th = _normalize(th_ref[...])
    dn = (((0,), (0,)), ((), ()))
    contrib = lax.dot_general(diff_ref[0], th, dn, preferred_element_type=jnp.float32)
    o_ref[0] = x_ref[0] + contrib * inv_p


def _recombine(x, diff, thetas, bn):
    B, N, D = x.shape
    P = thetas.shape[0]
    grid = (B, N // bn)
    return pl.pallas_call(
        functools.partial(_recomb_body, inv_p=1.0 / P),
        grid=grid,
        in_specs=[
            pl.BlockSpec((1, bn, D), lambda b, n: (b, n, 0)),
            pl.BlockSpec((1, P, bn), lambda b, n: (b, 0, n)),
            pl.BlockSpec((P, D), lambda b, n: (0, 0)),
        ],
        out_specs=pl.BlockSpec((1, bn, D), lambda b, n: (b, n, 0)),
        out_shape=jax.ShapeDtypeStruct((B, N, D), jnp.float32),
    )(x, diff, thetas)


def kernel(x_batch, y_batch, thetas, eps, n_projections):
    B, N, D = x_batch.shape
    P = thetas.shape[0]
    bn = 2048
    xp, xk, yk = _project(x_batch, y_batch, thetas, bn)
    diff = _sc_transport(
        xk.reshape(B * P, N), yk.reshape(B * P, N), xp.reshape(B * P, N)
    )
    return _recombine(x_batch, diff.reshape(B, P, N), thetas, bn)


# speedup vs baseline: 11.1510x; 1.0466x over previous
import functools

import jax
import jax.numpy as jnp
import numpy as np
from jax import lax
from jax.experimental import pallas as pl
from jax.experimental.pallas import tpu as pltpu
from jax.experimental.pallas import tpu_sc as plsc

L = 16
_MININT = np.int32(-2147483648)


def _normalize(th):
    n2 = jnp.sum(th * th, axis=1, keepdims=True)
    return th / jnp.maximum(jnp.sqrt(n2), 1e-12)


def _monotone(v):
    return jnp.where(v < 0, ~v, v ^ _MININT)


def _unmonotone_bits(m):
    return jnp.where(m < 0, m ^ _MININT, ~m)


def _proj_body(x_ref, y_ref, th_ref, xp_ref, xk_ref, yk_ref):
    th = _normalize(th_ref[...])
    dn = (((1,), (1,)), ((), ()))
    xp = lax.dot_general(th, x_ref[0], dn, preferred_element_type=jnp.float32)
    yp = lax.dot_general(th, y_ref[0], dn, preferred_element_type=jnp.float32)
    xp_ref[0] = xp
    xk_ref[0] = _monotone(lax.bitcast_convert_type(xp, jnp.int32))
    yk_ref[0] = _monotone(lax.bitcast_convert_type(yp, jnp.int32))


def _project(x, y, thetas, bn):
    B, N, D = x.shape
    P = thetas.shape[0]
    grid = (B, N // bn)
    xy_spec = pl.BlockSpec((1, bn, D), lambda b, n: (b, n, 0))
    th_spec = pl.BlockSpec((P, D), lambda b, n: (0, 0))
    out_spec = pl.BlockSpec((1, P, bn), lambda b, n: (b, 0, n))
    return pl.pallas_call(
        _proj_body,
        grid=grid,
        in_specs=[xy_spec, xy_spec, th_spec],
        out_specs=[out_spec, out_spec, out_spec],
        out_shape=[
            jax.ShapeDtypeStruct((B, P, N), jnp.float32),
            jax.ShapeDtypeStruct((B, P, N), jnp.int32),
            jax.ShapeDtypeStruct((B, P, N), jnp.int32),
        ],
    )(x, y, thetas)


def _scan_hist(hist, aux1, aux2):
    nhv = hist.shape[0] // L
    lane = lax.iota(jnp.int32, L)
    last = lane == (L - 1)

    @plsc.parallel_loop(0, nhv, unroll=8)
    def _(i):
        v = hist[pl.ds(i * L, L)]
        s = jnp.cumsum(v)
        hist[pl.ds(i * L, L)] = s - v
        plsc.store_scatter(aux1, [jnp.full((L,), i, jnp.int32)], s, mask=last)

    @plsc.parallel_loop(0, nhv // L, unroll=4)
    def _(i):
        v = aux1[pl.ds(i * L, L)]
        s = jnp.cumsum(v)
        aux1[pl.ds(i * L, L)] = s - v
        plsc.store_scatter(aux2, [jnp.full((L,), i, jnp.int32)], s, mask=last)

    a = aux2[pl.ds(0, L)]
    aux2[pl.ds(0, L)] = jnp.cumsum(a) - a

    @plsc.parallel_loop(0, nhv, unroll=8)
    def _(i):
        b1 = plsc.load_gather(aux1, [jnp.full((L,), i, jnp.int32)])
        b2 = plsc.load_gather(aux2, [jnp.full((L,), i >> 4, jnp.int32)])
        hist[pl.ds(i * L, L)] = hist[pl.ds(i * L, L)] + b1 + b2


def _radix_pass_xy(xk_s, xk_d, xv_s, xv_d, yk_s, yk_d, hx, hy, ax1, ax2, ay1, ay2, shift, chunk, lane):
    nhv = hx.shape[0] // L

    @plsc.parallel_loop(0, nhv, unroll=8)
    def _(i):
        hx[pl.ds(i * L, L)] = jnp.zeros((L,), jnp.int32)
        hy[pl.ds(i * L, L)] = jnp.zeros((L,), jnp.int32)

    ones = jnp.ones((L,), jnp.int32)

    @plsc.parallel_loop(0, chunk, unroll=4)
    def _(i):
        gidx = lane * chunk + i
        kx = plsc.load_gather(xk_s, [gidx])
        dx = lax.shift_right_logical(kx, shift) & 255
        plsc.addupdate_scatter(hx, [dx * L + lane], ones)
        ky = plsc.load_gather(yk_s, [gidx])
        dy = lax.shift_right_logical(ky, shift) & 255
        plsc.addupdate_scatter(hy, [dy * L + lane], ones)

    _scan_hist(hx, ax1, ax2)
    _scan_hist(hy, ay1, ay2)

    def perm_body(i, c):
        gidx = lane * chunk + i
        kx = plsc.load_gather(xk_s, [gidx])
        dx = lax.shift_right_logical(kx, shift) & 255
        hix = dx * L + lane
        ox = plsc.load_gather(hx, [hix])
        plsc.store_scatter(hx, [hix], ox + 1)
        plsc.store_scatter(xk_d, [ox], kx)
        plsc.store_scatter(xv_d, [ox], plsc.load_gather(xv_s, [gidx]))
        ky = plsc.load_gather(yk_s, [gidx])
        dy = lax.shift_right_logical(ky, shift) & 255
        hiy = dy * L + lane
        oy = plsc.load_gather(hy, [hiy])
        plsc.store_scatter(hy, [hiy], oy + 1)
        plsc.store_scatter(yk_d, [oy], ky)
        return c

    lax.fori_loop(0, chunk, perm_body, 0)


def _sc_transport_body(
    xk_hbm, yk_hbm, xp_hbm, out_hbm, xb, k0, k1, v0, v1, y0, y1, hx, hy, ax1, ax2, ay1, ay2
):
    nc = 2
    wid = lax.axis_index("s") * nc + lax.axis_index("c")
    rows = xk_hbm.shape[0]
    n = xk_hbm.shape[1]
    chunk = n // L
    nvec = n // L
    lane = lax.iota(jnp.int32, L)
    rows_per_w = rows // 32

    def row_body(t, c):
        r = wid * rows_per_w + t
        pltpu.sync_copy(xk_hbm.at[r], k0)
        pltpu.sync_copy(yk_hbm.at[r], y0)
        pltpu.sync_copy(xp_hbm.at[r], xb)

        @plsc.parallel_loop(0, nvec, unroll=8)
        def _(i):
            v0[pl.ds(i * L, L)] = lane + i * L

        for p in range(4):
            s, d = (k0, k1) if p % 2 == 0 else (k1, k0)
            sv, dv = (v0, v1) if p % 2 == 0 else (v1, v0)
            sy, dy = (y0, y1) if p % 2 == 0 else (y1, y0)
            _radix_pass_xy(s, d, sv, dv, sy, dy, hx, hy, ax1, ax2, ay1, ay2, 8 * p, chunk, lane)

        @plsc.parallel_loop(0, nvec, unroll=4)
        def _(i):
            sl = pl.ds(i * L, L)
            plsc.store_scatter(y1, [v0[sl]], _unmonotone_bits(y0[sl]))

        @plsc.parallel_loop(0, nvec, unroll=4)
        def _(i):
            sl = pl.ds(i * L, L)
            xb[sl] = lax.bitcast_convert_type(y1[sl], jnp.float32) - xb[sl]

        pltpu.sync_copy(xb, out_hbm.at[r])
        return c

    lax.fori_loop(0, rows_per_w, row_body, 0)


def _sc_transport(xk, yk, xp):
    R, N = xk.shape
    mesh = plsc.VectorSubcoreMesh(
        core_axis_name="c", subcore_axis_name="s", num_cores=2, num_subcores=16
    )
    f = pl.kernel(
        _sc_transport_body,
        out_type=jax.ShapeDtypeStruct((R, N), jnp.float32),
        mesh=mesh,
        compiler_params=pltpu.CompilerParams(needs_layout_passes=False),
        scratch_types=[
            pltpu.VMEM((N,), jnp.float32),
            pltpu.VMEM((N,), jnp.int32),
            pltpu.VMEM((N,), jnp.int32),
            pltpu.VMEM((N,), jnp.int32),
            pltpu.VMEM((N,), jnp.int32),
            pltpu.VMEM((N,), jnp.int32),
            pltpu.VMEM((N,), jnp.int32),
            pltpu.VMEM((256 * L,), jnp.int32),
            pltpu.VMEM((256 * L,), jnp.int32),
            pltpu.VMEM((256,), jnp.int32),
            pltpu.VMEM((L,), jnp.int32),
            pltpu.VMEM((256,), jnp.int32),
            pltpu.VMEM((L,), jnp.int32),
        ],
    )
    return f(xk, yk, xp)


def _recomb_body(x_ref, diff_ref, th_ref, o_ref, *, inv_p):
    th = _normalize(th_ref[...])
    dn = (((0,), (0,)), ((), ()))
    contrib = lax.dot_general(diff_ref[0], th, dn, preferred_element_type=jnp.float32)
    o_ref[0] = x_ref[0] + contrib * inv_p


def _recombine(x, diff, thetas, bn):
    B, N, D = x.shape
    P = thetas.shape[0]
    grid = (B, N // bn)
    return pl.pallas_call(
        functools.partial(_recomb_body, inv_p=1.0 / P),
        grid=grid,
        in_specs=[
            pl.BlockSpec((1, bn, D), lambda b, n: (b, n, 0)),
            pl.BlockSpec((1, P, bn), lambda b, n: (b, 0, n)),
            pl.BlockSpec((P, D), lambda b, n: (0, 0)),
        ],
        out_specs=pl.BlockSpec((1, bn, D), lambda b, n: (b, n, 0)),
        out_shape=jax.ShapeDtypeStruct((B, N, D), jnp.float32),
    )(x, diff, thetas)


def kernel(x_batch, y_batch, thetas, eps, n_projections):
    B, N, D = x_batch.shape
    P = thetas.shape[0]
    bn = 2048
    xp, xk, yk = _project(x_batch, y_batch, thetas, bn)
    diff = _sc_transport(
        xk.reshape(B * P, N), yk.reshape(B * P, N), xp.reshape(B * P, N)
    )
    return _recombine(x_batch, diff.reshape(B, P, N), thetas, bn)


# speedup vs baseline: 20.7178x; 1.8579x over previous
import functools

import jax
import jax.numpy as jnp
import numpy as np
from jax import lax
from jax.experimental import pallas as pl
from jax.experimental.pallas import tpu as pltpu
from jax.experimental.pallas import tpu_sc as plsc

L = 16
_MININT = np.int32(-2147483648)


def _normalize(th):
    n2 = jnp.sum(th * th, axis=1, keepdims=True)
    return th / jnp.maximum(jnp.sqrt(n2), 1e-12)


def _monotone(v):
    return jnp.where(v < 0, ~v, v ^ _MININT)


def _unmonotone_bits(m):
    return jnp.where(m < 0, m ^ _MININT, ~m)


def _proj_body(x_ref, y_ref, th_ref, xp_ref, xk_ref, yk_ref):
    th = _normalize(th_ref[...])
    dn = (((1,), (1,)), ((), ()))
    xp = lax.dot_general(th, x_ref[0], dn, preferred_element_type=jnp.float32)
    yp = lax.dot_general(th, y_ref[0], dn, preferred_element_type=jnp.float32)
    xp_ref[0] = xp
    xk_ref[0] = _monotone(lax.bitcast_convert_type(xp, jnp.int32))
    yk_ref[0] = _monotone(lax.bitcast_convert_type(yp, jnp.int32))


def _project(x, y, thetas, bn):
    B, N, D = x.shape
    P = thetas.shape[0]
    grid = (B, N // bn)
    xy_spec = pl.BlockSpec((1, bn, D), lambda b, n: (b, n, 0))
    th_spec = pl.BlockSpec((P, D), lambda b, n: (0, 0))
    out_spec = pl.BlockSpec((1, P, bn), lambda b, n: (b, 0, n))
    return pl.pallas_call(
        _proj_body,
        grid=grid,
        in_specs=[xy_spec, xy_spec, th_spec],
        out_specs=[out_spec, out_spec, out_spec],
        out_shape=[
            jax.ShapeDtypeStruct((B, P, N), jnp.float32),
            jax.ShapeDtypeStruct((B, P, N), jnp.int32),
            jax.ShapeDtypeStruct((B, P, N), jnp.int32),
        ],
    )(x, y, thetas)


def _scan_hist(hist, aux1, aux2):
    nhv = hist.shape[0] // L
    lane = lax.iota(jnp.int32, L)
    last = lane == (L - 1)

    @plsc.parallel_loop(0, nhv, unroll=8)
    def _(i):
        v = hist[pl.ds(i * L, L)]
        s = jnp.cumsum(v)
        hist[pl.ds(i * L, L)] = s - v
        plsc.store_scatter(aux1, [jnp.full((L,), i, jnp.int32)], s, mask=last)

    @plsc.parallel_loop(0, nhv // L, unroll=4)
    def _(i):
        v = aux1[pl.ds(i * L, L)]
        s = jnp.cumsum(v)
        aux1[pl.ds(i * L, L)] = s - v
        plsc.store_scatter(aux2, [jnp.full((L,), i, jnp.int32)], s, mask=last)

    a = aux2[pl.ds(0, L)]
    aux2[pl.ds(0, L)] = jnp.cumsum(a) - a

    @plsc.parallel_loop(0, nhv, unroll=8)
    def _(i):
        b1 = plsc.load_gather(aux1, [jnp.full((L,), i, jnp.int32)])
        b2 = plsc.load_gather(aux2, [jnp.full((L,), i >> 4, jnp.int32)])
        hist[pl.ds(i * L, L)] = hist[pl.ds(i * L, L)] + b1 + b2


def _t_of(o, chunk):
    return (o & (chunk - 1)) * L + lax.shift_right_logical(o, chunk.bit_length() - 1)


def _radix_pass_xy(xk_s, xk_d, xv_s, xv_d, yk_s, yk_d, hx, hy, ax1, ax2, ay1, ay2,
                   shift, chunk, lane, first, last):
    nhv = hx.shape[0] // L

    @plsc.parallel_loop(0, nhv, unroll=8)
    def _(i):
        hx[pl.ds(i * L, L)] = jnp.zeros((L,), jnp.int32)
        hy[pl.ds(i * L, L)] = jnp.zeros((L,), jnp.int32)

    ones = jnp.ones((L,), jnp.int32)

    @plsc.parallel_loop(0, chunk, unroll=4)
    def _(i):
        sl = pl.ds(i * L, L)
        dx = lax.shift_right_logical(xk_s[sl], shift) & 255
        plsc.addupdate_scatter(hx, [dx * L + lane], ones)
        dy = lax.shift_right_logical(yk_s[sl], shift) & 255
        plsc.addupdate_scatter(hy, [dy * L + lane], ones)

    _scan_hist(hx, ax1, ax2)
    _scan_hist(hy, ay1, ay2)

    def perm_body(i, c):
        sl = pl.ds(i * L, L)
        kx = xk_s[sl]
        dx = lax.shift_right_logical(kx, shift) & 255
        hix = dx * L + lane
        ox = plsc.load_gather(hx, [hix])
        plsc.store_scatter(hx, [hix], ox + 1)
        tox = _t_of(ox, chunk)
        if not last:
            plsc.store_scatter(xk_d, [tox], kx)
        vx = (lane + i * L) if first else xv_s[sl]
        plsc.store_scatter(xv_d, [tox], vx)
        ky = yk_s[sl]
        dy = lax.shift_right_logical(ky, shift) & 255
        hiy = dy * L + lane
        oy = plsc.load_gather(hy, [hiy])
        plsc.store_scatter(hy, [hiy], oy + 1)
        plsc.store_scatter(yk_d, [_t_of(oy, chunk)], ky)
        return c

    lax.fori_loop(0, chunk, perm_body, 0)


def _sc_transport_body(
    xk_hbm, yk_hbm, xp_hbm, out_hbm, xb, k0, k1, v0, v1, y0, y1, hx, hy, ax1, ax2, ay1, ay2
):
    nc = 2
    wid = lax.axis_index("s") * nc + lax.axis_index("c")
    rows = xk_hbm.shape[0]
    n = xk_hbm.shape[1]
    chunk = n // L
    nvec = n // L
    lane = lax.iota(jnp.int32, L)
    rows_per_w = rows // 32

    def row_body(t, c):
        r = wid * rows_per_w + t
        pltpu.sync_copy(xk_hbm.at[r], k0)
        pltpu.sync_copy(yk_hbm.at[r], y0)
        pltpu.sync_copy(xp_hbm.at[r], xb)

        for p in range(4):
            s, d = (k0, k1) if p % 2 == 0 else (k1, k0)
            sv, dv = (v0, v1) if p % 2 == 0 else (v1, v0)
            sy, dy = (y0, y1) if p % 2 == 0 else (y1, y0)
            _radix_pass_xy(s, d, sv, dv, sy, dy, hx, hy, ax1, ax2, ay1, ay2,
                           8 * p, chunk, lane, p == 0, p == 3)

        @plsc.parallel_loop(0, nvec, unroll=4)
        def _(i):
            sl = pl.ds(i * L, L)
            plsc.store_scatter(y1, [v0[sl]], _unmonotone_bits(y0[sl]))

        @plsc.parallel_loop(0, nvec, unroll=4)
        def _(i):
            sl = pl.ds(i * L, L)
            xb[sl] = lax.bitcast_convert_type(y1[sl], jnp.float32) - xb[sl]

        pltpu.sync_copy(xb, out_hbm.at[r])
        return c

    lax.fori_loop(0, rows_per_w, row_body, 0)


def _sc_transport(xk, yk, xp):
    R, N = xk.shape
    mesh = plsc.VectorSubcoreMesh(
        core_axis_name="c", subcore_axis_name="s", num_cores=2, num_subcores=16
    )
    f = pl.kernel(
        _sc_transport_body,
        out_type=jax.ShapeDtypeStruct((R, N), jnp.float32),
        mesh=mesh,
        compiler_params=pltpu.CompilerParams(needs_layout_passes=False),
        scratch_types=[
            pltpu.VMEM((N,), jnp.float32),
            pltpu.VMEM((N,), jnp.int32),
            pltpu.VMEM((N,), jnp.int32),
            pltpu.VMEM((N,), jnp.int32),
            pltpu.VMEM((N,), jnp.int32),
            pltpu.VMEM((N,), jnp.int32),
            pltpu.VMEM((N,), jnp.int32),
            pltpu.VMEM((256 * L,), jnp.int32),
            pltpu.VMEM((256 * L,), jnp.int32),
            pltpu.VMEM((256,), jnp.int32),
            pltpu.VMEM((L,), jnp.int32),
            pltpu.VMEM((256,), jnp.int32),
            pltpu.VMEM((L,), jnp.int32),
        ],
    )
    return f(xk, yk, xp)


def _recomb_body(x_ref, diff_ref, th_ref, o_ref, *, inv_p):
    th = _normalize(th_ref[...])
    dn = (((0,), (0,)), ((), ()))
    contrib = lax.dot_general(diff_ref[0], th, dn, preferred_element_type=jnp.float32)
    o_ref[0] = x_ref[0] + contrib * inv_p


def _recombine(x, diff, thetas, bn):
    B, N, D = x.shape
    P = thetas.shape[0]
    grid = (B, N // bn)
    return pl.pallas_call(
        functools.partial(_recomb_body, inv_p=1.0 / P),
        grid=grid,
        in_specs=[
            pl.BlockSpec((1, bn, D), lambda b, n: (b, n, 0)),
            pl.BlockSpec((1, P, bn), lambda b, n: (b, 0, n)),
            pl.BlockSpec((P, D), lambda b, n: (0, 0)),
        ],
        out_specs=pl.BlockSpec((1, bn, D), lambda b, n: (b, n, 0)),
        out_shape=jax.ShapeDtypeStruct((B, N, D), jnp.float32),
    )(x, diff, thetas)


def kernel(x_batch, y_batch, thetas, eps, n_projections):
    B, N, D = x_batch.shape
    P = thetas.shape[0]
    bn = 2048
    xp, xk, yk = _project(x_batch, y_batch, thetas, bn)
    diff = _sc_transport(
        xk.reshape(B * P, N), yk.reshape(B * P, N), xp.reshape(B * P, N)
    )
    return _recombine(x_batch, diff.reshape(B, P, N), thetas, bn)


# speedup vs baseline: 36.1760x; 1.7461x over previous
import functools

import jax
import jax.numpy as jnp
import numpy as np
from jax import lax
from jax.experimental import pallas as pl
from jax.experimental.pallas import tpu as pltpu
from jax.experimental.pallas import tpu_sc as plsc

L = 16
_MININT = np.int32(-2147483648)


def _normalize(th):
    n2 = jnp.sum(th * th, axis=1, keepdims=True)
    return th / jnp.maximum(jnp.sqrt(n2), 1e-12)


def _monotone(v):
    return jnp.where(v < 0, ~v, v ^ _MININT)


def _unmonotone_bits(m):
    return jnp.where(m < 0, m ^ _MININT, ~m)


def _proj_body(x_ref, y_ref, th_ref, xp_ref, xk_ref, yk_ref):
    th = _normalize(th_ref[...])
    dn = (((1,), (1,)), ((), ()))
    xp = lax.dot_general(th, x_ref[0], dn, preferred_element_type=jnp.float32)
    yp = lax.dot_general(th, y_ref[0], dn, preferred_element_type=jnp.float32)
    xp_ref[0] = xp
    xk_ref[0] = _monotone(lax.bitcast_convert_type(xp, jnp.int32))
    yk_ref[0] = _monotone(lax.bitcast_convert_type(yp, jnp.int32))


def _project(x, y, thetas, bn):
    B, N, D = x.shape
    P = thetas.shape[0]
    grid = (B, N // bn)
    xy_spec = pl.BlockSpec((1, bn, D), lambda b, n: (b, n, 0))
    th_spec = pl.BlockSpec((P, D), lambda b, n: (0, 0))
    out_spec = pl.BlockSpec((1, P, bn), lambda b, n: (b, 0, n))
    return pl.pallas_call(
        _proj_body,
        grid=grid,
        in_specs=[xy_spec, xy_spec, th_spec],
        out_specs=[out_spec, out_spec, out_spec],
        out_shape=[
            jax.ShapeDtypeStruct((B, P, N), jnp.float32),
            jax.ShapeDtypeStruct((B, P, N), jnp.int32),
            jax.ShapeDtypeStruct((B, P, N), jnp.int32),
        ],
    )(x, y, thetas)


def _scan_hist(hist, aux1, aux2):
    nhv = hist.shape[0] // L
    lane = lax.iota(jnp.int32, L)
    last = lane == (L - 1)

    @plsc.parallel_loop(0, nhv, unroll=8)
    def _(i):
        v = hist[pl.ds(i * L, L)]
        s = jnp.cumsum(v)
        hist[pl.ds(i * L, L)] = s - v
        plsc.store_scatter(aux1, [jnp.full((L,), i, jnp.int32)], s, mask=last)

    @plsc.parallel_loop(0, nhv // L, unroll=4)
    def _(i):
        v = aux1[pl.ds(i * L, L)]
        s = jnp.cumsum(v)
        aux1[pl.ds(i * L, L)] = s - v
        plsc.store_scatter(aux2, [jnp.full((L,), i, jnp.int32)], s, mask=last)

    a = aux2[pl.ds(0, L)]
    aux2[pl.ds(0, L)] = jnp.cumsum(a) - a

    @plsc.parallel_loop(0, nhv, unroll=8)
    def _(i):
        b1 = plsc.load_gather(aux1, [jnp.full((L,), i, jnp.int32)])
        b2 = plsc.load_gather(aux2, [jnp.full((L,), i >> 4, jnp.int32)])
        hist[pl.ds(i * L, L)] = hist[pl.ds(i * L, L)] + b1 + b2


def _t_of(o, chunk):
    return (o & (chunk - 1)) * L + lax.shift_right_logical(o, chunk.bit_length() - 1)


def _radix_pass_xy(xk_s, xk_d, xv_s, xv_d, yk_s, yk_d, hx, hy, ax1, ax2, ay1, ay2,
                   shift, chunk, lane, first, last):
    nhv = hx.shape[0] // L

    @plsc.parallel_loop(0, nhv, unroll=8)
    def _(i):
        hx[pl.ds(i * L, L)] = jnp.zeros((L,), jnp.int32)
        hy[pl.ds(i * L, L)] = jnp.zeros((L,), jnp.int32)

    ones = jnp.ones((L,), jnp.int32)

    @plsc.parallel_loop(0, chunk, unroll=4)
    def _(i):
        sl = pl.ds(i * L, L)
        dx = lax.shift_right_logical(xk_s[sl], shift) & 255
        plsc.addupdate_scatter(hx, [dx * L + lane], ones)
        dy = lax.shift_right_logical(yk_s[sl], shift) & 255
        plsc.addupdate_scatter(hy, [dy * L + lane], ones)

    _scan_hist(hx, ax1, ax2)
    _scan_hist(hy, ay1, ay2)

    def load_kv(i):
        sl = pl.ds(i * L, L)
        vx = (lane + i * L) if first else xv_s[sl]
        kx = xk_s[sl]
        ky = yk_s[sl]
        hix = (lax.shift_right_logical(kx, shift) & 255) * L + lane
        hiy = (lax.shift_right_logical(ky, shift) & 255) * L + lane
        return kx, vx, ky, hix, hiy

    def perm_body(i, c):
        kx, vx, ky, hix, hiy = c
        ox = plsc.load_gather(hx, [hix])
        oy = plsc.load_gather(hy, [hiy])
        nc_ = load_kv(jnp.minimum(i + 1, chunk - 1))
        plsc.store_scatter(hx, [hix], ox + 1)
        plsc.store_scatter(hy, [hiy], oy + 1)
        tox = _t_of(ox, chunk)
        toy = _t_of(oy, chunk)
        if not last:
            plsc.store_scatter(xk_d, [tox], kx)
        plsc.store_scatter(xv_d, [tox], vx)
        plsc.store_scatter(yk_d, [toy], ky)
        return nc_

    lax.fori_loop(0, chunk, perm_body, load_kv(0))


def _sc_transport_body(
    xk_hbm, yk_hbm, xp_hbm, out_hbm, xb, k0, k1, v0, v1, y0, y1, hx, hy, ax1, ax2, ay1, ay2
):
    nc = 2
    wid = lax.axis_index("s") * nc + lax.axis_index("c")
    rows = xk_hbm.shape[0]
    n = xk_hbm.shape[1]
    chunk = n // L
    nvec = n // L
    lane = lax.iota(jnp.int32, L)
    rows_per_w = rows // 32

    def row_body(t, c):
        r = wid * rows_per_w + t
        pltpu.sync_copy(xk_hbm.at[r], k0)
        pltpu.sync_copy(yk_hbm.at[r], y0)
        pltpu.sync_copy(xp_hbm.at[r], xb)

        for p in range(4):
            s, d = (k0, k1) if p % 2 == 0 else (k1, k0)
            sv, dv = (v0, v1) if p % 2 == 0 else (v1, v0)
            sy, dy = (y0, y1) if p % 2 == 0 else (y1, y0)
            _radix_pass_xy(s, d, sv, dv, sy, dy, hx, hy, ax1, ax2, ay1, ay2,
                           8 * p, chunk, lane, p == 0, p == 3)

        @plsc.parallel_loop(0, nvec, unroll=4)
        def _(i):
            sl = pl.ds(i * L, L)
            plsc.store_scatter(y1, [v0[sl]], _unmonotone_bits(y0[sl]))

        @plsc.parallel_loop(0, nvec, unroll=4)
        def _(i):
            sl = pl.ds(i * L, L)
            xb[sl] = lax.bitcast_convert_type(y1[sl], jnp.float32) - xb[sl]

        pltpu.sync_copy(xb, out_hbm.at[r])
        return c

    lax.fori_loop(0, rows_per_w, row_body, 0)


def _sc_transport(xk, yk, xp):
    R, N = xk.shape
    mesh = plsc.VectorSubcoreMesh(
        core_axis_name="c", subcore_axis_name="s", num_cores=2, num_subcores=16
    )
    f = pl.kernel(
        _sc_transport_body,
        out_type=jax.ShapeDtypeStruct((R, N), jnp.float32),
        mesh=mesh,
        compiler_params=pltpu.CompilerParams(needs_layout_passes=False),
        scratch_types=[
            pltpu.VMEM((N,), jnp.float32),
            pltpu.VMEM((N,), jnp.int32),
            pltpu.VMEM((N,), jnp.int32),
            pltpu.VMEM((N,), jnp.int32),
            pltpu.VMEM((N,), jnp.int32),
            pltpu.VMEM((N,), jnp.int32),
            pltpu.VMEM((N,), jnp.int32),
            pltpu.VMEM((256 * L,), jnp.int32),
            pltpu.VMEM((256 * L,), jnp.int32),
            pltpu.VMEM((256,), jnp.int32),
            pltpu.VMEM((L,), jnp.int32),
            pltpu.VMEM((256,), jnp.int32),
            pltpu.VMEM((L,), jnp.int32),
        ],
    )
    return f(xk, yk, xp)


def _recomb_body(x_ref, diff_ref, th_ref, o_ref, *, inv_p):
    th = _normalize(th_ref[...])
    dn = (((0,), (0,)), ((), ()))
    contrib = lax.dot_general(diff_ref[0], th, dn, preferred_element_type=jnp.float32)
    o_ref[0] = x_ref[0] + contrib * inv_p


def _recombine(x, diff, thetas, bn):
    B, N, D = x.shape
    P = thetas.shape[0]
    grid = (B, N // bn)
    return pl.pallas_call(
        functools.partial(_recomb_body, inv_p=1.0 / P),
        grid=grid,
        in_specs=[
            pl.BlockSpec((1, bn, D), lambda b, n: (b, n, 0)),
            pl.BlockSpec((1, P, bn), lambda b, n: (b, 0, n)),
            pl.BlockSpec((P, D), lambda b, n: (0, 0)),
        ],
        out_specs=pl.BlockSpec((1, bn, D), lambda b, n: (b, n, 0)),
        out_shape=jax.ShapeDtypeStruct((B, N, D), jnp.float32),
    )(x, diff, thetas)


def kernel(x_batch, y_batch, thetas, eps, n_projections):
    B, N, D = x_batch.shape
    P = thetas.shape[0]
    bn = 2048
    xp, xk, yk = _project(x_batch, y_batch, thetas, bn)
    diff = _sc_transport(
        xk.reshape(B * P, N), yk.reshape(B * P, N), xp.reshape(B * P, N)
    )
    return _recombine(x_batch, diff.reshape(B, P, N), thetas, bn)


# speedup vs baseline: 50.4800x; 1.3954x over previous
import functools

import jax
import jax.numpy as jnp
import numpy as np
from jax import lax
from jax.experimental import pallas as pl
from jax.experimental.pallas import tpu as pltpu
from jax.experimental.pallas import tpu_sc as plsc

L = 16
S = 1
_MININT = np.int32(-2147483648)


def _normalize(th):
    n2 = jnp.sum(th * th, axis=1, keepdims=True)
    return th / jnp.maximum(jnp.sqrt(n2), 1e-12)


def _monotone(v):
    return jnp.where(v < 0, ~v, v ^ _MININT)


def _unmonotone_bits(m):
    return jnp.where(m < 0, m ^ _MININT, ~m)


def _proj_body(x_ref, y_ref, th_ref, xpb_ref, xk_ref, yk_ref):
    th = _normalize(th_ref[...])
    dn = (((1,), (1,)), ((), ()))
    xp = lax.dot_general(th, x_ref[0], dn, preferred_element_type=jnp.float32)
    yp = lax.dot_general(th, y_ref[0], dn, preferred_element_type=jnp.float32)
    xpb = lax.bitcast_convert_type(xp, jnp.int32)
    xpb_ref[0] = xpb
    xk_ref[0] = _monotone(xpb)
    yk_ref[0] = _monotone(lax.bitcast_convert_type(yp, jnp.int32))


def _project(x, y, thetas, bn):
    B, N, D = x.shape
    P = thetas.shape[0]
    grid = (B, N // bn)
    xy_spec = pl.BlockSpec((1, bn, D), lambda b, n: (b, n, 0))
    th_spec = pl.BlockSpec((P, D), lambda b, n: (0, 0))
    out_spec = pl.BlockSpec((1, P, bn), lambda b, n: (b, 0, n))
    shape = jax.ShapeDtypeStruct((B, P, N), jnp.int32)
    return pl.pallas_call(
        _proj_body,
        grid=grid,
        in_specs=[xy_spec, xy_spec, th_spec],
        out_specs=[out_spec, out_spec, out_spec],
        out_shape=[shape, shape, shape],
    )(x, y, thetas)


def _scan_hist(hist, aux1, aux2):
    nhv = hist.shape[0] // L
    n2 = nhv // L
    lane = lax.iota(jnp.int32, L)
    last = lane == (L - 1)

    @plsc.parallel_loop(0, nhv, unroll=8)
    def _(i):
        v = hist[pl.ds(i * L, L)]
        s = jnp.cumsum(v)
        hist[pl.ds(i * L, L)] = s - v
        plsc.store_scatter(aux1, [jnp.full((L,), i, jnp.int32)], s, mask=last)

    @plsc.parallel_loop(0, n2, unroll=4)
    def _(i):
        v = aux1[pl.ds(i * L, L)]
        s = jnp.cumsum(v)
        aux1[pl.ds(i * L, L)] = s - v
        plsc.store_scatter(aux2, [jnp.full((L,), i, jnp.int32)], s, mask=last)

    def c_body(i, carry):
        v = aux2[pl.ds(i * L, L)]
        s = jnp.cumsum(v)
        aux2[pl.ds(i * L, L)] = s - v + carry
        return carry + jnp.sum(v)

    lax.fori_loop(0, n2 // L if n2 > L else 1, c_body, jnp.int32(0))

    @plsc.parallel_loop(0, nhv, unroll=8)
    def _(i):
        b1 = plsc.load_gather(aux1, [jnp.full((L,), i, jnp.int32)])
        b2 = plsc.load_gather(aux2, [jnp.full((L,), i >> 4, jnp.int32)])
        hist[pl.ds(i * L, L)] = hist[pl.ds(i * L, L)] + b1 + b2


def _t_of(o, chunk):
    return (o & (chunk - 1)) * L + lax.shift_right_logical(o, chunk.bit_length() - 1)


def _radix_pass_xy(xk_s, xk_d, xv_s, xv_d, yk_s, yk_d, hx, hy, ax1, ax2, ay1, ay2,
                   shift, chunk, lane, first, last):
    sub = chunk // S

    @plsc.parallel_loop(0, hx.shape[0] // L, unroll=8)
    def _(i):
        hx[pl.ds(i * L, L)] = jnp.zeros((L,), jnp.int32)
        hy[pl.ds(i * L, L)] = jnp.zeros((L,), jnp.int32)

    ones = jnp.ones((L,), jnp.int32)
    lane_s = lane * S

    @plsc.parallel_loop(0, sub, unroll=2)
    def _(q):
        for s in range(S):
            sl = pl.ds((s * sub + q) * L, L)
            dx = lax.shift_right_logical(xk_s[sl], shift) & 255
            plsc.addupdate_scatter(hx, [dx * (L * S) + lane_s + s], ones)
            dy = lax.shift_right_logical(yk_s[sl], shift) & 255
            plsc.addupdate_scatter(hy, [dy * (L * S) + lane_s + s], ones)

    _scan_hist(hx, ax1, ax2)
    _scan_hist(hy, ay1, ay2)

    def load_kv(q):
        out = []
        for s in range(S):
            i = s * sub + q
            sl = pl.ds(i * L, L)
            kx = xk_s[sl]
            ky = yk_s[sl]
            vx = (lane + i * L) if first else xv_s[sl]
            hix = (lax.shift_right_logical(kx, shift) & 255) * (L * S) + lane_s + s
            hiy = (lax.shift_right_logical(ky, shift) & 255) * (L * S) + lane_s + s
            out.append((kx, vx, ky, hix, hiy))
        return tuple(out)

    def perm_body(q, c):
        ox = [None] * S
        oy = [None] * S
        for s in range(S):
            kx, vx, ky, hix, hiy = c[s]
            ox[s] = plsc.load_gather(hx, [hix])
            oy[s] = plsc.load_gather(hy, [hiy])
        nc_ = load_kv(jnp.minimum(q + 1, sub - 1))
        for s in range(S):
            kx, vx, ky, hix, hiy = c[s]
            plsc.store_scatter(hx, [hix], ox[s] + 1)
            plsc.store_scatter(hy, [hiy], oy[s] + 1)
        for s in range(S):
            kx, vx, ky, hix, hiy = c[s]
            tox = _t_of(ox[s], chunk)
            toy = _t_of(oy[s], chunk)
            if not last:
                plsc.store_scatter(xk_d, [tox], kx)
            plsc.store_scatter(xv_d, [tox], vx)
            plsc.store_scatter(yk_d, [toy], ky)
        return nc_

    lax.fori_loop(0, sub, perm_body, load_kv(0))


def _sc_transport_body(
    xk_hbm, yk_hbm, xpb_hbm, out_hbm, k0, k1, v0, v1, y0, y1, hx, hy, ax1, ax2, ay1, ay2
):
    nc = 2
    wid = lax.axis_index("s") * nc + lax.axis_index("c")
    rows = xk_hbm.shape[0]
    n = xk_hbm.shape[1]
    chunk = n // L
    nvec = n // L
    lane = lax.iota(jnp.int32, L)
    rows_per_w = rows // 32

    def row_body(t, c):
        r = wid * rows_per_w + t
        pltpu.sync_copy(xk_hbm.at[r], k0)
        pltpu.sync_copy(yk_hbm.at[r], y0)

        for p in range(4):
            s, d = (k0, k1) if p % 2 == 0 else (k1, k0)
            sv, dv = (v0, v1) if p % 2 == 0 else (v1, v0)
            sy, dy = (y0, y1) if p % 2 == 0 else (y1, y0)
            _radix_pass_xy(s, d, sv, dv, sy, dy, hx, hy, ax1, ax2, ay1, ay2,
                           8 * p, chunk, lane, p == 0, p == 3)

        pltpu.sync_copy(xpb_hbm.at[r], k1)

        @plsc.parallel_loop(0, nvec, unroll=4)
        def _(i):
            sl = pl.ds(i * L, L)
            idx = v0[sl]
            ysf = lax.bitcast_convert_type(_unmonotone_bits(y0[sl]), jnp.float32)
            xpf = lax.bitcast_convert_type(plsc.load_gather(k1, [idx]), jnp.float32)
            plsc.store_scatter(k0, [idx], lax.bitcast_convert_type(ysf - xpf, jnp.int32))

        pltpu.sync_copy(k0, out_hbm.at[r])
        return c

    lax.fori_loop(0, rows_per_w, row_body, 0)


def _sc_transport(xk, yk, xpb):
    R, N = xk.shape
    mesh = plsc.VectorSubcoreMesh(
        core_axis_name="c", subcore_axis_name="s", num_cores=2, num_subcores=16
    )
    nbins = 256 * L * S
    f = pl.kernel(
        _sc_transport_body,
        out_type=jax.ShapeDtypeStruct((R, N), jnp.int32),
        mesh=mesh,
        compiler_params=pltpu.CompilerParams(needs_layout_passes=False),
        scratch_types=[
            pltpu.VMEM((N,), jnp.int32),
            pltpu.VMEM((N,), jnp.int32),
            pltpu.VMEM((N,), jnp.int32),
            pltpu.VMEM((N,), jnp.int32),
            pltpu.VMEM((N,), jnp.int32),
            pltpu.VMEM((N,), jnp.int32),
            pltpu.VMEM((nbins,), jnp.int32),
            pltpu.VMEM((nbins,), jnp.int32),
            pltpu.VMEM((nbins // L,), jnp.int32),
            pltpu.VMEM((max(nbins // L // L, L),), jnp.int32),
            pltpu.VMEM((nbins // L,), jnp.int32),
            pltpu.VMEM((max(nbins // L // L, L),), jnp.int32),
        ],
    )
    return f(xk, yk, xpb)


def _recomb_body(x_ref, diff_ref, th_ref, o_ref, *, inv_p):
    th = _normalize(th_ref[...])
    diff = lax.bitcast_convert_type(diff_ref[0], jnp.float32)
    dn = (((0,), (0,)), ((), ()))
    contrib = lax.dot_general(diff, th, dn, preferred_element_type=jnp.float32)
    o_ref[0] = x_ref[0] + contrib * inv_p


def _recombine(x, diffb, thetas, bn):
    B, N, D = x.shape
    P = thetas.shape[0]
    grid = (B, N // bn)
    return pl.pallas_call(
        functools.partial(_recomb_body, inv_p=1.0 / P),
        grid=grid,
        in_specs=[
            pl.BlockSpec((1, bn, D), lambda b, n: (b, n, 0)),
            pl.BlockSpec((1, P, bn), lambda b, n: (b, 0, n)),
            pl.BlockSpec((P, D), lambda b, n: (0, 0)),
        ],
        out_specs=pl.BlockSpec((1, bn, D), lambda b, n: (b, n, 0)),
        out_shape=jax.ShapeDtypeStruct((B, N, D), jnp.float32),
    )(x, diffb, thetas)


def kernel(x_batch, y_batch, thetas, eps, n_projections):
    B, N, D = x_batch.shape
    P = thetas.shape[0]
    bn = 2048
    xpb, xk, yk = _project(x_batch, y_batch, thetas, bn)
    diffb = _sc_transport(
        xk.reshape(B * P, N), yk.reshape(B * P, N), xpb.reshape(B * P, N)
    )
    return _recombine(x_batch, diffb.reshape(B, P, N), thetas, bn)


# speedup vs baseline: 54.1986x; 1.0737x over previous
import functools

import jax
import jax.numpy as jnp
import numpy as np
from jax import lax
from jax.experimental import pallas as pl
from jax.experimental.pallas import tpu as pltpu
from jax.experimental.pallas import tpu_sc as plsc

L = 16
S = 1
_MININT = np.int32(-2147483648)


def _normalize(th):
    n2 = jnp.sum(th * th, axis=1, keepdims=True)
    return th / jnp.maximum(jnp.sqrt(n2), 1e-12)


def _monotone(v):
    return jnp.where(v < 0, ~v, v ^ _MININT)


def _unmonotone_bits(m):
    return jnp.where(m < 0, m ^ _MININT, ~m)


def _proj_body(x_ref, y_ref, th_ref, xpb_ref, xk_ref, yk_ref):
    th = _normalize(th_ref[...])
    dn = (((1,), (1,)), ((), ()))
    xp = lax.dot_general(th, x_ref[0], dn, preferred_element_type=jnp.float32)
    yp = lax.dot_general(th, y_ref[0], dn, preferred_element_type=jnp.float32)
    xpb = lax.bitcast_convert_type(xp, jnp.int32)
    xpb_ref[0] = xpb
    xk_ref[0] = _monotone(xpb)
    yk_ref[0] = _monotone(lax.bitcast_convert_type(yp, jnp.int32))


def _project(x, y, thetas, bn):
    B, N, D = x.shape
    P = thetas.shape[0]
    grid = (B, N // bn)
    xy_spec = pl.BlockSpec((1, bn, D), lambda b, n: (b, n, 0))
    th_spec = pl.BlockSpec((P, D), lambda b, n: (0, 0))
    out_spec = pl.BlockSpec((1, P, bn), lambda b, n: (b, 0, n))
    shape = jax.ShapeDtypeStruct((B, P, N), jnp.int32)
    return pl.pallas_call(
        _proj_body,
        grid=grid,
        in_specs=[xy_spec, xy_spec, th_spec],
        out_specs=[out_spec, out_spec, out_spec],
        out_shape=[shape, shape, shape],
    )(x, y, thetas)


def _scan_hist(hist, aux1, aux2):
    nhv = hist.shape[0] // L
    n2 = nhv // L
    lane = lax.iota(jnp.int32, L)
    last = lane == (L - 1)

    @plsc.parallel_loop(0, nhv, unroll=8)
    def _(i):
        v = hist[pl.ds(i * L, L)]
        s = jnp.cumsum(v)
        hist[pl.ds(i * L, L)] = s - v
        plsc.store_scatter(aux1, [jnp.full((L,), i, jnp.int32)], s, mask=last)

    @plsc.parallel_loop(0, n2, unroll=4)
    def _(i):
        v = aux1[pl.ds(i * L, L)]
        s = jnp.cumsum(v)
        aux1[pl.ds(i * L, L)] = s - v
        plsc.store_scatter(aux2, [jnp.full((L,), i, jnp.int32)], s, mask=last)

    def c_body(i, carry):
        v = aux2[pl.ds(i * L, L)]
        s = jnp.cumsum(v)
        aux2[pl.ds(i * L, L)] = s - v + carry
        return carry + jnp.sum(v)

    lax.fori_loop(0, n2 // L if n2 > L else 1, c_body, jnp.int32(0))

    @plsc.parallel_loop(0, nhv, unroll=8)
    def _(i):
        b1 = plsc.load_gather(aux1, [jnp.full((L,), i, jnp.int32)])
        b2 = plsc.load_gather(aux2, [jnp.full((L,), i >> 4, jnp.int32)])
        hist[pl.ds(i * L, L)] = hist[pl.ds(i * L, L)] + b1 + b2


def _t_of(o, chunk):
    return (o & (chunk - 1)) * L + lax.shift_right_logical(o, chunk.bit_length() - 1)


def _radix_pass_xy(xk_s, xk_d, xv_s, xv_d, yk_s, yk_d, hx, hy, ax1, ax2, ay1, ay2,
                   shift, chunk, lane, first, last):
    sub = chunk // S

    @plsc.parallel_loop(0, hx.shape[0] // L, unroll=8)
    def _(i):
        hx[pl.ds(i * L, L)] = jnp.zeros((L,), jnp.int32)
        hy[pl.ds(i * L, L)] = jnp.zeros((L,), jnp.int32)

    ones = jnp.ones((L,), jnp.int32)
    lane_s = lane * S

    @plsc.parallel_loop(0, sub, unroll=2)
    def _(q):
        for s in range(S):
            sl = pl.ds((s * sub + q) * L, L)
            dx = lax.shift_right_logical(xk_s[sl], shift) & 255
            plsc.addupdate_scatter(hx, [dx * (L * S) + lane_s + s], ones)
            dy = lax.shift_right_logical(yk_s[sl], shift) & 255
            plsc.addupdate_scatter(hy, [dy * (L * S) + lane_s + s], ones)

    _scan_hist(hx, ax1, ax2)
    _scan_hist(hy, ay1, ay2)

    U = 4

    def load_kv(q):
        out = []
        for s in range(S):
            for u in range(U):
                i = s * sub + q * U + u
                sl = pl.ds(i * L, L)
                kx = xk_s[sl]
                ky = yk_s[sl]
                vx = (lane + i * L) if first else xv_s[sl]
                hix = (lax.shift_right_logical(kx, shift) & 255) * (L * S) + lane_s + s
                hiy = (lax.shift_right_logical(ky, shift) & 255) * (L * S) + lane_s + s
                out.append((kx, vx, ky, hix, hiy))
        return tuple(out)

    def perm_body(q, c):
        n_ = S * U
        ox = [plsc.load_gather(hx, [c[k][3]]) for k in range(n_)]
        oy = [plsc.load_gather(hy, [c[k][4]]) for k in range(n_)]
        nc_ = load_kv(jnp.minimum(q + 1, sub // U - 1))
        for s in range(S):
            for u in range(1, U):
                k = s * U + u
                for j in range(s * U, k):
                    ox[k] = ox[k] + jnp.where(c[k][3] == c[j][3], 1, 0)
                    oy[k] = oy[k] + jnp.where(c[k][4] == c[j][4], 1, 0)
        for k in range(n_):
            plsc.store_scatter(hx, [c[k][3]], ox[k] + 1)
            plsc.store_scatter(hy, [c[k][4]], oy[k] + 1)
        for k in range(n_):
            kx, vx, ky, _, _ = c[k]
            tox = _t_of(ox[k], chunk)
            toy = _t_of(oy[k], chunk)
            if not last:
                plsc.store_scatter(xk_d, [tox], kx)
            plsc.store_scatter(xv_d, [tox], vx)
            plsc.store_scatter(yk_d, [toy], ky)
        return nc_

    lax.fori_loop(0, sub // U, perm_body, load_kv(0))


def _sc_transport_body(
    xk_hbm, yk_hbm, xpb_hbm, out_hbm, k0, k1, v0, v1, y0, y1, hx, hy, ax1, ax2, ay1, ay2
):
    nc = 2
    wid = lax.axis_index("s") * nc + lax.axis_index("c")
    rows = xk_hbm.shape[0]
    n = xk_hbm.shape[1]
    chunk = n // L
    nvec = n // L
    lane = lax.iota(jnp.int32, L)
    rows_per_w = rows // 32

    def row_body(t, c):
        r = wid * rows_per_w + t
        pltpu.sync_copy(xk_hbm.at[r], k0)
        pltpu.sync_copy(yk_hbm.at[r], y0)

        for p in range(4):
            s, d = (k0, k1) if p % 2 == 0 else (k1, k0)
            sv, dv = (v0, v1) if p % 2 == 0 else (v1, v0)
            sy, dy = (y0, y1) if p % 2 == 0 else (y1, y0)
            _radix_pass_xy(s, d, sv, dv, sy, dy, hx, hy, ax1, ax2, ay1, ay2,
                           8 * p, chunk, lane, p == 0, p == 3)

        pltpu.sync_copy(xpb_hbm.at[r], k1)

        @plsc.parallel_loop(0, nvec, unroll=4)
        def _(i):
            sl = pl.ds(i * L, L)
            idx = v0[sl]
            ysf = lax.bitcast_convert_type(_unmonotone_bits(y0[sl]), jnp.float32)
            xpf = lax.bitcast_convert_type(plsc.load_gather(k1, [idx]), jnp.float32)
            plsc.store_scatter(k0, [idx], lax.bitcast_convert_type(ysf - xpf, jnp.int32))

        pltpu.sync_copy(k0, out_hbm.at[r])
        return c

    lax.fori_loop(0, rows_per_w, row_body, 0)


def _sc_transport(xk, yk, xpb):
    R, N = xk.shape
    mesh = plsc.VectorSubcoreMesh(
        core_axis_name="c", subcore_axis_name="s", num_cores=2, num_subcores=16
    )
    nbins = 256 * L * S
    f = pl.kernel(
        _sc_transport_body,
        out_type=jax.ShapeDtypeStruct((R, N), jnp.int32),
        mesh=mesh,
        compiler_params=pltpu.CompilerParams(needs_layout_passes=False),
        scratch_types=[
            pltpu.VMEM((N,), jnp.int32),
            pltpu.VMEM((N,), jnp.int32),
            pltpu.VMEM((N,), jnp.int32),
            pltpu.VMEM((N,), jnp.int32),
            pltpu.VMEM((N,), jnp.int32),
            pltpu.VMEM((N,), jnp.int32),
            pltpu.VMEM((nbins,), jnp.int32),
            pltpu.VMEM((nbins,), jnp.int32),
            pltpu.VMEM((nbins // L,), jnp.int32),
            pltpu.VMEM((max(nbins // L // L, L),), jnp.int32),
            pltpu.VMEM((nbins // L,), jnp.int32),
            pltpu.VMEM((max(nbins // L // L, L),), jnp.int32),
        ],
    )
    return f(xk, yk, xpb)


def _recomb_body(x_ref, diff_ref, th_ref, o_ref, *, inv_p):
    th = _normalize(th_ref[...])
    diff = lax.bitcast_convert_type(diff_ref[0], jnp.float32)
    dn = (((0,), (0,)), ((), ()))
    contrib = lax.dot_general(diff, th, dn, preferred_element_type=jnp.float32)
    o_ref[0] = x_ref[0] + contrib * inv_p


def _recombine(x, diffb, thetas, bn):
    B, N, D = x.shape
    P = thetas.shape[0]
    grid = (B, N // bn)
    return pl.pallas_call(
        functools.partial(_recomb_body, inv_p=1.0 / P),
        grid=grid,
        in_specs=[
            pl.BlockSpec((1, bn, D), lambda b, n: (b, n, 0)),
            pl.BlockSpec((1, P, bn), lambda b, n: (b, 0, n)),
            pl.BlockSpec((P, D), lambda b, n: (0, 0)),
        ],
        out_specs=pl.BlockSpec((1, bn, D), lambda b, n: (b, n, 0)),
        out_shape=jax.ShapeDtypeStruct((B, N, D), jnp.float32),
    )(x, diffb, thetas)


def kernel(x_batch, y_batch, thetas, eps, n_projections):
    B, N, D = x_batch.shape
    P = thetas.shape[0]
    bn = 4096
    xpb, xk, yk = _project(x_batch, y_batch, thetas, bn)
    diffb = _sc_transport(
        xk.reshape(B * P, N), yk.reshape(B * P, N), xpb.reshape(B * P, N)
    )
    return _recombine(x_batch, diffb.reshape(B, P, N), thetas, bn)
